# baseline (speedup 1.0000x reference)
import functools

import jax
import jax.numpy as jnp
from jax import lax
from jax.experimental import pallas as pl
from jax.experimental.pallas import tpu as pltpu
from jax.experimental.pallas import tpu_sc as plsc

NNODES = 100000
NEDGES = 1600000
NPAD = 100352
NP8 = NPAD // 8
NTILES = 32
SLICE = NPAD // 16
CHUNK_E = 768
NOUT = 66
PAIRS = (NOUT - 2) // 2
EPAD = NTILES * CHUNK_E * NOUT
AGG_NOUT0 = 86
AGG_NOUT1 = 46

_MESH = plsc.VectorSubcoreMesh(
    core_axis_name="c", subcore_axis_name="s", num_cores=2, num_subcores=16)

_SC_PARAMS = pltpu.CompilerParams(use_tc_tiling_on_sc=False)


@functools.partial(
    pl.kernel,
    out_type=[jax.ShapeDtypeStruct((NPAD, 16), jnp.float32),
              jax.ShapeDtypeStruct((NPAD, 16), jnp.float32)],
    mesh=_MESH,
    scratch_types=[
        pltpu.VMEM((CHUNK_E,), jnp.int32),
        pltpu.VMEM((CHUNK_E,), jnp.int32),
        pltpu.VMEM((CHUNK_E, 16), jnp.float32),
        pltpu.VMEM_SHARED((NPAD, 16), jnp.float32),
        pltpu.SemaphoreType.DMA,
        pltpu.SemaphoreType.DMA,
    ],
    compiler_params=_SC_PARAMS,
)
def _sc_deg(dst_hbm, ones_hbm, zeros_hbm, out0_hbm, out1_hbm, db0, db1, obuf, acc,
            ssem, isem):
    c = lax.axis_index("c")
    s = lax.axis_index("s")
    wid = s * 2 + c
    r0 = s * SLICE
    pltpu.sync_copy(zeros_hbm.at[pl.ds(r0, SLICE)], acc.at[pl.ds(r0, SLICE)])
    pltpu.sync_copy(ones_hbm, obuf)
    plsc.subcore_barrier()
    base = wid * (CHUNK_E * NOUT)

    def load_idx(i, db):
        pltpu.async_copy(dst_hbm.at[pl.ds(base + i * CHUNK_E, CHUNK_E)],
                         db, isem)

    def wait_idx():
        pltpu.make_async_copy(dst_hbm.at[pl.ds(0, CHUNK_E)], db0, isem).wait()

    def fire_scatters(db):
        pltpu.async_copy(obuf, acc.at[db], ssem, add=True)

    def wait_scatters():
        pltpu.make_async_copy(ones_hbm, obuf, ssem).wait()

    load_idx(0, db0)
    wait_idx()
    load_idx(1, db1)
    fire_scatters(db0)

    @pl.loop(0, PAIRS)
    def _pair(k):
        i1 = 2 * k + 1
        wait_idx()
        wait_scatters()
        load_idx(i1 + 1, db0)
        fire_scatters(db1)
        wait_idx()
        wait_scatters()
        load_idx(i1 + 2, db1)
        fire_scatters(db0)

    wait_idx()
    wait_scatters()
    fire_scatters(db1)
    wait_scatters()

    plsc.subcore_barrier()

    @pl.when(c == 0)
    def _wb0():
        pltpu.sync_copy(acc.at[pl.ds(r0, SLICE)], out0_hbm.at[pl.ds(r0, SLICE)])

    @pl.when(c == 1)
    def _wb1():
        pltpu.sync_copy(acc.at[pl.ds(r0, SLICE)], out1_hbm.at[pl.ds(r0, SLICE)])


@functools.partial(
    pl.kernel,
    out_type=[jax.ShapeDtypeStruct((NPAD, 16), jnp.float32),
              jax.ShapeDtypeStruct((NPAD, 16), jnp.float32)],
    mesh=_MESH,
    scratch_types=[
        pltpu.VMEM((CHUNK_E,), jnp.int32),
        pltpu.VMEM((CHUNK_E,), jnp.int32),
        pltpu.VMEM((CHUNK_E,), jnp.int32),
        pltpu.VMEM((CHUNK_E,), jnp.int32),
        pltpu.VMEM((CHUNK_E, 16), jnp.float32),
        pltpu.VMEM((CHUNK_E, 16), jnp.float32),
        pltpu.VMEM_SHARED((NPAD, 16), jnp.float32),
        pltpu.SemaphoreType.DMA,
        pltpu.SemaphoreType.DMA,
        pltpu.SemaphoreType.DMA,
    ],
    compiler_params=_SC_PARAMS,
)
def _sc_agg(src_hbm, dst_hbm, g_hbm, out0_hbm, out1_hbm, sb0, db0, sb1, db1, rb0, rb1,
            acc, gsem, ssem, isem):
    c = lax.axis_index("c")
    s = lax.axis_index("s")
    wid = s * 2 + c
    r0 = s * SLICE
    pltpu.sync_copy(g_hbm.at[pl.ds(r0, SLICE)], acc.at[pl.ds(r0, SLICE)])
    plsc.subcore_barrier()
    base = jnp.where(c == 0, s * AGG_NOUT0,
                     16 * AGG_NOUT0 + s * AGG_NOUT1) * CHUNK_E
    pairs = jnp.where(c == 0, (AGG_NOUT0 - 2) // 2, (AGG_NOUT1 - 2) // 2)

    def load_idx(i, sb, db):
        pltpu.async_copy(src_hbm.at[pl.ds(base + i * CHUNK_E, CHUNK_E)],
                         sb, isem)
        pltpu.async_copy(dst_hbm.at[pl.ds(base + i * CHUNK_E, CHUNK_E)],
                         db, isem)

    def wait_idx():
        for _ in range(2):
            pltpu.make_async_copy(src_hbm.at[pl.ds(0, CHUNK_E)], sb0,
                                  isem).wait()

    def fire_gathers(sb, rb):
        pltpu.async_copy(g_hbm.at[sb], rb, gsem)

    def wait_gathers(rb):
        pltpu.make_async_copy(g_hbm.at[pl.ds(0, CHUNK_E)], rb, gsem).wait()

    def fire_scatters(db, rb):
        pltpu.async_copy(rb, acc.at[db], ssem, add=True)

    def wait_scatters(rb):
        pltpu.make_async_copy(g_hbm.at[pl.ds(0, CHUNK_E)], rb, ssem).wait()

    load_idx(0, sb0, db0)
    wait_idx()
    fire_gathers(sb0, rb0)
    load_idx(1, sb1, db1)
    wait_gathers(rb0)
    fire_scatters(db0, rb0)

    @pl.loop(0, pairs)
    def _pair(k):
        i1 = 2 * k + 1
        wait_idx()
        fire_gathers(sb1, rb1)
        wait_scatters(rb0)
        load_idx(i1 + 1, sb0, db0)
        wait_gathers(rb1)
        fire_scatters(db1, rb1)
        wait_idx()
        fire_gathers(sb0, rb0)
        wait_scatters(rb1)
        load_idx(i1 + 2, sb1, db1)
        wait_gathers(rb0)
        fire_scatters(db0, rb0)

    wait_idx()
    fire_gathers(sb1, rb1)
    wait_scatters(rb0)
    wait_gathers(rb1)
    fire_scatters(db1, rb1)
    wait_scatters(rb1)

    plsc.subcore_barrier()

    @pl.when(c == 0)
    def _wb0():
        pltpu.sync_copy(acc.at[pl.ds(r0, SLICE)], out0_hbm.at[pl.ds(r0, SLICE)])

    @pl.when(c == 1)
    def _wb1():
        pltpu.sync_copy(acc.at[pl.ds(r0, SLICE)], out1_hbm.at[pl.ds(r0, SLICE)])


_BR = 1568


def _tc_prep_body(d0, d1, xp, dinv_out, g1_out):
    dinv = 1.0 / jnp.sqrt(d0[...] + d1[...] + 1.0)
    dinv_out[...] = dinv
    g1_out[...] = dinv * xp[...]


def _tc_layer_body(wblk, brep, y0, y1, g, dinv, gnext_out):
    z = dinv[...] * (y0[...] + y1[...] - g[...])
    h = jnp.tanh(
        jnp.dot(z, wblk[...], preferred_element_type=jnp.float32,
                precision=lax.Precision.HIGHEST) + brep[...])
    gnext_out[...] = dinv[...] * h


_BR2 = 5000


def _tc_final_body(w3, b3, wc, bc2, y0, y1, g, dinv, h_out, o_out):
    z = dinv[...] * (y0[...] + y1[...] - g[...])
    h = jnp.tanh(
        jnp.dot(z, w3[...], preferred_element_type=jnp.float32,
                precision=lax.Precision.HIGHEST) + b3[...])
    h_out[...] = h
    o_out[...] = (
        jnp.dot(h, wc[...], preferred_element_type=jnp.float32,
                precision=lax.Precision.HIGHEST) + bc2[...])


def _row_spec(br, ncols, offset_blocks=0):
    return pl.BlockSpec((br, ncols), lambda i, o=offset_blocks: (i + o, 0))


def _fixed_spec(shape):
    return pl.BlockSpec(shape, lambda i: (0, 0))


def kernel(x, edge_index, W1, b1, Wh0, bh0, Wh1, bh1, W3, b3, Wc, bc):
    f32 = jnp.float32
    src = edge_index[0]
    dst = edge_index[1]
    pad = EPAD - NEDGES
    padv = jnp.full((pad,), NNODES, jnp.int32)
    src1d = jnp.concatenate([src, padv])
    dst1d = jnp.concatenate([dst, padv])
    xpad = jnp.zeros((NPAD, 16), f32).at[:NNODES, :3].set(x)
    zeros16 = jnp.zeros((NPAD, 16), f32)
    ones_payload = jnp.ones((CHUNK_E, 16), f32)

    def pad16(w, b):
        wp = jnp.zeros((16, 16), f32).at[:w.shape[0], :w.shape[1]].set(w)
        bp = jnp.zeros((16,), f32).at[:b.shape[0]].set(b)
        return (jnp.kron(jnp.eye(8, dtype=f32), wp),
                jnp.tile(bp, 8).reshape(1, 128))

    wblk1, brep1 = pad16(W1, b1)
    wblkh0, breph0 = pad16(Wh0, bh0)
    wblkh1, breph1 = pad16(Wh1, bh1)
    w3p = jnp.zeros((16, 120), f32).at[:15, :].set(W3)
    b3p = b3.reshape(1, 120)
    wcp = Wc
    bcp = bc.reshape(1, 2)

    d0, d1 = _sc_deg(dst1d, ones_payload, zeros16)

    nblk = NP8 // _BR
    dinv128, g128 = pl.pallas_call(
        _tc_prep_body,
        grid=(nblk,),
        in_specs=[_row_spec(_BR, 128), _row_spec(_BR, 128),
                  _row_spec(_BR, 128)],
        out_specs=[_row_spec(_BR, 128), _row_spec(_BR, 128)],
        out_shape=[jax.ShapeDtypeStruct((NP8, 128), f32),
                   jax.ShapeDtypeStruct((NP8, 128), f32)],
    )(d0.reshape(NP8, 128), d1.reshape(NP8, 128), xpad.reshape(NP8, 128))

    for wblk, brep in ((wblk1, brep1), (wblkh0, breph0), (wblkh1, breph1)):
        y0, y1 = _sc_agg(src1d, dst1d, g128.reshape(NPAD, 16))
        g128 = pl.pallas_call(
            _tc_layer_body,
            grid=(nblk,),
            in_specs=[_fixed_spec((128, 128)), _fixed_spec((1, 128)),
                      _row_spec(_BR, 128), _row_spec(_BR, 128),
                      _row_spec(_BR, 128), _row_spec(_BR, 128)],
            out_specs=_row_spec(_BR, 128),
            out_shape=jax.ShapeDtypeStruct((NP8, 128), f32),
        )(wblk, brep, y0.reshape(NP8, 128), y1.reshape(NP8, 128),
          g128, dinv128)

    y0, y1 = _sc_agg(src1d, dst1d, g128.reshape(NPAD, 16))
    nblk2 = NNODES // _BR2
    h, out = pl.pallas_call(
        _tc_final_body,
        grid=(nblk2,),
        in_specs=[_fixed_spec((16, 120)), _fixed_spec((1, 120)),
                  _fixed_spec((120, 2)), _fixed_spec((1, 2)),
                  _row_spec(_BR2, 16), _row_spec(_BR2, 16),
                  _row_spec(_BR2, 16), _row_spec(_BR2, 16)],
        out_specs=[_row_spec(_BR2, 120), _row_spec(_BR2, 2)],
        out_shape=[jax.ShapeDtypeStruct((NNODES, 120), f32),
                   jax.ShapeDtypeStruct((NNODES, 2), f32)],
    )(w3p, b3p, wcp, bcp, y0, y1, g128.reshape(NPAD, 16),
      dinv128.reshape(NPAD, 16))

    return (out, h)


# speedup vs baseline: 1.0662x; 1.0662x over previous
import functools

import jax
import jax.numpy as jnp
from jax import lax
from jax.experimental import pallas as pl
from jax.experimental.pallas import tpu as pltpu
from jax.experimental.pallas import tpu_sc as plsc

NNODES = 100000
NEDGES = 1600000
NPAD = 100352
NP8 = NPAD // 8
NTILES = 32
SLICE = NPAD // 16
CHUNK_E = 768
NOUT = 66
PAIRS = (NOUT - 2) // 2
EPAD = NTILES * CHUNK_E * NOUT
AGG_NOUT0 = 86
AGG_NOUT1 = 46

_MESH = plsc.VectorSubcoreMesh(
    core_axis_name="c", subcore_axis_name="s", num_cores=2, num_subcores=16)

_SC_PARAMS = pltpu.CompilerParams(use_tc_tiling_on_sc=False)


@functools.partial(
    pl.kernel,
    out_type=jax.ShapeDtypeStruct((2 * NPAD, 16), jnp.float32),
    mesh=_MESH,
    scratch_types=[
        pltpu.VMEM((CHUNK_E,), jnp.int32),
        pltpu.VMEM((CHUNK_E,), jnp.int32),
        pltpu.VMEM((CHUNK_E, 16), jnp.float32),
        pltpu.VMEM_SHARED((NPAD, 16), jnp.float32),
        pltpu.SemaphoreType.DMA,
        pltpu.SemaphoreType.DMA,
    ],
    compiler_params=_SC_PARAMS,
)
def _sc_deg(dst_hbm, ones_hbm, zeros_hbm, out_hbm, db0, db1, obuf, acc,
            ssem, isem):
    c = lax.axis_index("c")
    s = lax.axis_index("s")
    wid = s * 2 + c
    r0 = s * SLICE
    pltpu.sync_copy(zeros_hbm.at[pl.ds(r0, SLICE)], acc.at[pl.ds(r0, SLICE)])
    pltpu.sync_copy(ones_hbm, obuf)
    plsc.subcore_barrier()
    base = wid * (CHUNK_E * NOUT)

    def load_idx(i, db):
        pltpu.async_copy(dst_hbm.at[pl.ds(base + i * CHUNK_E, CHUNK_E)],
                         db, isem)

    def wait_idx():
        pltpu.make_async_copy(dst_hbm.at[pl.ds(0, CHUNK_E)], db0, isem).wait()

    def fire_scatters(db):
        pltpu.async_copy(obuf, acc.at[db], ssem, add=True)

    def wait_scatters():
        pltpu.make_async_copy(ones_hbm, obuf, ssem).wait()

    load_idx(0, db0)
    wait_idx()
    load_idx(1, db1)
    fire_scatters(db0)

    @pl.loop(0, PAIRS)
    def _pair(k):
        i1 = 2 * k + 1
        wait_idx()
        wait_scatters()
        load_idx(i1 + 1, db0)
        fire_scatters(db1)
        wait_idx()
        wait_scatters()
        load_idx(i1 + 2, db1)
        fire_scatters(db0)

    wait_idx()
    wait_scatters()
    fire_scatters(db1)
    wait_scatters()

    plsc.subcore_barrier()
    pltpu.sync_copy(acc.at[pl.ds(r0, SLICE)],
                    out_hbm.at[pl.ds(c * NPAD + r0, SLICE)])


@functools.partial(
    pl.kernel,
    out_type=jax.ShapeDtypeStruct((2 * NPAD, 16), jnp.float32),
    mesh=_MESH,
    scratch_types=[
        pltpu.VMEM((CHUNK_E,), jnp.int32),
        pltpu.VMEM((CHUNK_E,), jnp.int32),
        pltpu.VMEM((CHUNK_E,), jnp.int32),
        pltpu.VMEM((CHUNK_E,), jnp.int32),
        pltpu.VMEM((CHUNK_E, 16), jnp.float32),
        pltpu.VMEM((CHUNK_E, 16), jnp.float32),
        pltpu.VMEM_SHARED((NPAD, 16), jnp.float32),
        pltpu.SemaphoreType.DMA,
        pltpu.SemaphoreType.DMA,
        pltpu.SemaphoreType.DMA,
    ],
    compiler_params=_SC_PARAMS,
)
def _sc_agg(src_hbm, dst_hbm, g_hbm, out_hbm, sb0, db0, sb1, db1, rb0, rb1,
            acc, gsem, ssem, isem):
    c = lax.axis_index("c")
    s = lax.axis_index("s")
    wid = s * 2 + c
    r0 = s * SLICE
    pltpu.sync_copy(g_hbm.at[pl.ds(r0, SLICE)], acc.at[pl.ds(r0, SLICE)])
    plsc.subcore_barrier()
    base = jnp.where(c == 0, s * AGG_NOUT0,
                     16 * AGG_NOUT0 + s * AGG_NOUT1) * CHUNK_E
    pairs = jnp.where(c == 0, (AGG_NOUT0 - 2) // 2, (AGG_NOUT1 - 2) // 2)

    def load_idx(i, sb, db):
        pltpu.async_copy(src_hbm.at[pl.ds(base + i * CHUNK_E, CHUNK_E)],
                         sb, isem)
        pltpu.async_copy(dst_hbm.at[pl.ds(base + i * CHUNK_E, CHUNK_E)],
                         db, isem)

    def wait_idx():
        for _ in range(2):
            pltpu.make_async_copy(src_hbm.at[pl.ds(0, CHUNK_E)], sb0,
                                  isem).wait()

    def fire_gathers(sb, rb):
        pltpu.async_copy(g_hbm.at[sb], rb, gsem)

    def wait_gathers(rb):
        pltpu.make_async_copy(g_hbm.at[pl.ds(0, CHUNK_E)], rb, gsem).wait()

    def fire_scatters(db, rb):
        pltpu.async_copy(rb, acc.at[db], ssem, add=True)

    def wait_scatters(rb):
        pltpu.make_async_copy(g_hbm.at[pl.ds(0, CHUNK_E)], rb, ssem).wait()

    load_idx(0, sb0, db0)
    wait_idx()
    fire_gathers(sb0, rb0)
    load_idx(1, sb1, db1)
    wait_gathers(rb0)
    fire_scatters(db0, rb0)

    @pl.loop(0, pairs)
    def _pair(k):
        i1 = 2 * k + 1
        wait_idx()
        fire_gathers(sb1, rb1)
        wait_scatters(rb0)
        load_idx(i1 + 1, sb0, db0)
        wait_gathers(rb1)
        fire_scatters(db1, rb1)
        wait_idx()
        fire_gathers(sb0, rb0)
        wait_scatters(rb1)
        load_idx(i1 + 2, sb1, db1)
        wait_gathers(rb0)
        fire_scatters(db0, rb0)

    wait_idx()
    fire_gathers(sb1, rb1)
    wait_scatters(rb0)
    wait_gathers(rb1)
    fire_scatters(db1, rb1)
    wait_scatters(rb1)

    plsc.subcore_barrier()
    pltpu.sync_copy(acc.at[pl.ds(r0, SLICE)],
                    out_hbm.at[pl.ds(c * NPAD + r0, SLICE)])


_BR = 1568


def _tc_prep_body(d0, d1, xp, dinv_out, g1_out):
    dinv = 1.0 / jnp.sqrt(d0[...] + d1[...] + 1.0)
    dinv_out[...] = dinv
    g1_out[...] = dinv * xp[...]


def _tc_layer_body(wblk, brep, y0, y1, g, dinv, gnext_out):
    z = dinv[...] * (y0[...] + y1[...] - g[...])
    h = jnp.tanh(
        jnp.dot(z, wblk[...], preferred_element_type=jnp.float32,
                precision=lax.Precision.HIGHEST) + brep[...])
    gnext_out[...] = dinv[...] * h


_BR2 = 3584


def _tc_final_body(w3, b3, wc, bc2, y0, y1, g, dinv, h_out, o_out):
    z = dinv[...] * (y0[...] + y1[...] - g[...])
    h = jnp.tanh(
        jnp.dot(z, w3[...], preferred_element_type=jnp.float32,
                precision=lax.Precision.HIGHEST) + b3[...])
    h_out[...] = h
    o_out[...] = (
        jnp.dot(h, wc[...], preferred_element_type=jnp.float32,
                precision=lax.Precision.HIGHEST) + bc2[...])


def _row_spec(br, ncols, offset_blocks=0):
    return pl.BlockSpec((br, ncols), lambda i, o=offset_blocks: (i + o, 0))


def _fixed_spec(shape):
    return pl.BlockSpec(shape, lambda i: (0, 0))


def kernel(x, edge_index, W1, b1, Wh0, bh0, Wh1, bh1, W3, b3, Wc, bc):
    f32 = jnp.float32
    src = edge_index[0]
    dst = edge_index[1]
    pad = EPAD - NEDGES
    padv = jnp.full((pad,), NNODES, jnp.int32)
    src1d = jnp.concatenate([src, padv])
    dst1d = jnp.concatenate([dst, padv])
    xpad = jnp.zeros((NPAD, 16), f32).at[:NNODES, :3].set(x)
    zeros16 = jnp.zeros((NPAD, 16), f32)
    ones_payload = jnp.ones((CHUNK_E, 16), f32)

    def pad16(w, b):
        wp = jnp.zeros((16, 16), f32).at[:w.shape[0], :w.shape[1]].set(w)
        bp = jnp.zeros((16,), f32).at[:b.shape[0]].set(b)
        return (jnp.kron(jnp.eye(8, dtype=f32), wp),
                jnp.tile(bp, 8).reshape(1, 128))

    wblk1, brep1 = pad16(W1, b1)
    wblkh0, breph0 = pad16(Wh0, bh0)
    wblkh1, breph1 = pad16(Wh1, bh1)
    w3p = jnp.zeros((16, 120), f32).at[:15, :].set(W3)
    b3p = b3.reshape(1, 120)
    wcp = Wc
    bcp = bc.reshape(1, 2)

    degp = _sc_deg(dst1d, ones_payload, zeros16)
    degp128 = degp.reshape(2 * NP8, 128)

    nblk = NP8 // _BR
    dinv128, g128 = pl.pallas_call(
        _tc_prep_body,
        grid=(nblk,),
        in_specs=[_row_spec(_BR, 128), _row_spec(_BR, 128, nblk),
                  _row_spec(_BR, 128)],
        out_specs=[_row_spec(_BR, 128), _row_spec(_BR, 128)],
        out_shape=[jax.ShapeDtypeStruct((NP8, 128), f32),
                   jax.ShapeDtypeStruct((NP8, 128), f32)],
    )(degp128, degp128, xpad.reshape(NP8, 128))

    for wblk, brep in ((wblk1, brep1), (wblkh0, breph0), (wblkh1, breph1)):
        y = _sc_agg(src1d, dst1d, g128.reshape(NPAD, 16))
        y128 = y.reshape(2 * NP8, 128)
        g128 = pl.pallas_call(
            _tc_layer_body,
            grid=(nblk,),
            in_specs=[_fixed_spec((128, 128)), _fixed_spec((1, 128)),
                      _row_spec(_BR, 128), _row_spec(_BR, 128, nblk),
                      _row_spec(_BR, 128), _row_spec(_BR, 128)],
            out_specs=_row_spec(_BR, 128),
            out_shape=jax.ShapeDtypeStruct((NP8, 128), f32),
        )(wblk, brep, y128, y128, g128, dinv128)

    y = _sc_agg(src1d, dst1d, g128.reshape(NPAD, 16))
    nblk2 = NPAD // _BR2
    hfull, ofull = pl.pallas_call(
        _tc_final_body,
        grid=(nblk2,),
        in_specs=[_fixed_spec((16, 120)), _fixed_spec((1, 120)),
                  _fixed_spec((120, 2)), _fixed_spec((1, 2)),
                  _row_spec(_BR2, 16), _row_spec(_BR2, 16, nblk2),
                  _row_spec(_BR2, 16), _row_spec(_BR2, 16)],
        out_specs=[_row_spec(_BR2, 120), _row_spec(_BR2, 2)],
        out_shape=[jax.ShapeDtypeStruct((NPAD, 120), f32),
                   jax.ShapeDtypeStruct((NPAD, 2), f32)],
    )(w3p, b3p, wcp, bcp, y, y, g128.reshape(NPAD, 16),
      dinv128.reshape(NPAD, 16))

    return (ofull[:NNODES], hfull[:NNODES])


# speedup vs baseline: 1.1012x; 1.0328x over previous
import functools

import jax
import jax.numpy as jnp
from jax import lax
from jax.experimental import pallas as pl
from jax.experimental.pallas import tpu as pltpu
from jax.experimental.pallas import tpu_sc as plsc

NNODES = 100000
NEDGES = 1600000
NPAD = 100352
NP8 = NPAD // 8
NTILES = 32
SLICE = NPAD // 16
CHUNK_E = 768
NOUT = 66
PAIRS = (NOUT - 2) // 2
EPAD = NTILES * CHUNK_E * NOUT
AGG_NOUT0 = 92
AGG_NOUT1 = 40

_MESH = plsc.VectorSubcoreMesh(
    core_axis_name="c", subcore_axis_name="s", num_cores=2, num_subcores=16)

_SC_PARAMS = pltpu.CompilerParams(use_tc_tiling_on_sc=False)


@functools.partial(
    pl.kernel,
    out_type=jax.ShapeDtypeStruct((2 * NPAD, 16), jnp.float32),
    mesh=_MESH,
    scratch_types=[
        pltpu.VMEM((CHUNK_E,), jnp.int32),
        pltpu.VMEM((CHUNK_E,), jnp.int32),
        pltpu.VMEM((CHUNK_E, 16), jnp.float32),
        pltpu.VMEM_SHARED((NPAD, 16), jnp.float32),
        pltpu.SemaphoreType.DMA,
        pltpu.SemaphoreType.DMA,
    ],
    compiler_params=_SC_PARAMS,
)
def _sc_deg(dst_hbm, ones_hbm, zeros_hbm, out_hbm, db0, db1, obuf, acc,
            ssem, isem):
    c = lax.axis_index("c")
    s = lax.axis_index("s")
    wid = s * 2 + c
    r0 = s * SLICE
    pltpu.sync_copy(zeros_hbm.at[pl.ds(r0, SLICE)], acc.at[pl.ds(r0, SLICE)])
    pltpu.sync_copy(ones_hbm, obuf)
    plsc.subcore_barrier()
    base = wid * (CHUNK_E * NOUT)

    def load_idx(i, db):
        pltpu.async_copy(dst_hbm.at[pl.ds(base + i * CHUNK_E, CHUNK_E)],
                         db, isem)

    def wait_idx():
        pltpu.make_async_copy(dst_hbm.at[pl.ds(0, CHUNK_E)], db0, isem).wait()

    def fire_scatters(db):
        pltpu.async_copy(obuf, acc.at[db], ssem, add=True)

    def wait_scatters():
        pltpu.make_async_copy(ones_hbm, obuf, ssem).wait()

    load_idx(0, db0)
    wait_idx()
    load_idx(1, db1)
    fire_scatters(db0)

    @pl.loop(0, PAIRS)
    def _pair(k):
        i1 = 2 * k + 1
        wait_idx()
        wait_scatters()
        load_idx(i1 + 1, db0)
        fire_scatters(db1)
        wait_idx()
        wait_scatters()
        load_idx(i1 + 2, db1)
        fire_scatters(db0)

    wait_idx()
    wait_scatters()
    fire_scatters(db1)
    wait_scatters()

    plsc.subcore_barrier()
    pltpu.sync_copy(acc.at[pl.ds(r0, SLICE)],
                    out_hbm.at[pl.ds(c * NPAD + r0, SLICE)])


@functools.partial(
    pl.kernel,
    out_type=jax.ShapeDtypeStruct((2 * NPAD, 16), jnp.float32),
    mesh=_MESH,
    scratch_types=[
        pltpu.VMEM((CHUNK_E,), jnp.int32),
        pltpu.VMEM((CHUNK_E,), jnp.int32),
        pltpu.VMEM((CHUNK_E,), jnp.int32),
        pltpu.VMEM((CHUNK_E,), jnp.int32),
        pltpu.VMEM((CHUNK_E, 16), jnp.float32),
        pltpu.VMEM((CHUNK_E, 16), jnp.float32),
        pltpu.VMEM_SHARED((NPAD, 16), jnp.float32),
        pltpu.SemaphoreType.DMA,
        pltpu.SemaphoreType.DMA,
        pltpu.SemaphoreType.DMA,
    ],
    compiler_params=_SC_PARAMS,
)
def _sc_agg(src_hbm, dst_hbm, g_hbm, out_hbm, sb0, db0, sb1, db1, rb0, rb1,
            acc, gsem, ssem, isem):
    c = lax.axis_index("c")
    s = lax.axis_index("s")
    wid = s * 2 + c
    r0 = s * SLICE
    pltpu.sync_copy(g_hbm.at[pl.ds(r0, SLICE)], acc.at[pl.ds(r0, SLICE)])
    plsc.subcore_barrier()
    base = jnp.where(c == 0, s * AGG_NOUT0,
                     16 * AGG_NOUT0 + s * AGG_NOUT1) * CHUNK_E
    pairs = jnp.where(c == 0, (AGG_NOUT0 - 2) // 2, (AGG_NOUT1 - 2) // 2)

    def load_idx(i, sb, db):
        pltpu.async_copy(src_hbm.at[pl.ds(base + i * CHUNK_E, CHUNK_E)],
                         sb, isem)
        pltpu.async_copy(dst_hbm.at[pl.ds(base + i * CHUNK_E, CHUNK_E)],
                         db, isem)

    def wait_idx():
        for _ in range(2):
            pltpu.make_async_copy(src_hbm.at[pl.ds(0, CHUNK_E)], sb0,
                                  isem).wait()

    def fire_gathers(sb, rb):
        pltpu.async_copy(g_hbm.at[sb], rb, gsem)

    def wait_gathers(rb):
        pltpu.make_async_copy(g_hbm.at[pl.ds(0, CHUNK_E)], rb, gsem).wait()

    def fire_scatters(db, rb):
        pltpu.async_copy(rb, acc.at[db], ssem, add=True)

    def wait_scatters(rb):
        pltpu.make_async_copy(g_hbm.at[pl.ds(0, CHUNK_E)], rb, ssem).wait()

    load_idx(0, sb0, db0)
    wait_idx()
    fire_gathers(sb0, rb0)
    load_idx(1, sb1, db1)
    wait_gathers(rb0)
    fire_scatters(db0, rb0)

    @pl.loop(0, pairs)
    def _pair(k):
        i1 = 2 * k + 1
        wait_idx()
        fire_gathers(sb1, rb1)
        wait_scatters(rb0)
        load_idx(i1 + 1, sb0, db0)
        wait_gathers(rb1)
        fire_scatters(db1, rb1)
        wait_idx()
        fire_gathers(sb0, rb0)
        wait_scatters(rb1)
        load_idx(i1 + 2, sb1, db1)
        wait_gathers(rb0)
        fire_scatters(db0, rb0)

    wait_idx()
    fire_gathers(sb1, rb1)
    wait_scatters(rb0)
    wait_gathers(rb1)
    fire_scatters(db1, rb1)
    wait_scatters(rb1)

    plsc.subcore_barrier()
    pltpu.sync_copy(acc.at[pl.ds(r0, SLICE)],
                    out_hbm.at[pl.ds(c * NPAD + r0, SLICE)])


_BR = 1568


def _tc_prep_body(d0, d1, xp, dinv_out, g1_out):
    dinv = 1.0 / jnp.sqrt(d0[...] + d1[...] + 1.0)
    dinv_out[...] = dinv
    g1_out[...] = dinv * xp[...]


def _tc_layer_body(wblk, brep, y0, y1, g, dinv, gnext_out):
    z = dinv[...] * (y0[...] + y1[...] - g[...])
    h = jnp.tanh(
        jnp.dot(z, wblk[...], preferred_element_type=jnp.float32,
                precision=lax.Precision.HIGHEST) + brep[...])
    gnext_out[...] = dinv[...] * h


_BR2 = 3584


def _tc_final_body(w3, b3, wc, bc2, y0, y1, g, dinv, h_out, o_out):
    z = dinv[...] * (y0[...] + y1[...] - g[...])
    h = jnp.tanh(
        jnp.dot(z, w3[...], preferred_element_type=jnp.float32,
                precision=lax.Precision.HIGHEST) + b3[...])
    h_out[...] = h
    o_out[...] = (
        jnp.dot(h, wc[...], preferred_element_type=jnp.float32,
                precision=lax.Precision.HIGHEST) + bc2[...])


def _row_spec(br, ncols, offset_blocks=0):
    return pl.BlockSpec((br, ncols), lambda i, o=offset_blocks: (i + o, 0))


def _fixed_spec(shape):
    return pl.BlockSpec(shape, lambda i: (0, 0))


def kernel(x, edge_index, W1, b1, Wh0, bh0, Wh1, bh1, W3, b3, Wc, bc):
    f32 = jnp.float32
    src = edge_index[0]
    dst = edge_index[1]
    pad = EPAD - NEDGES
    padv = jnp.full((pad,), NNODES, jnp.int32)
    src1d = jnp.concatenate([src, padv])
    dst1d = jnp.concatenate([dst, padv])
    xpad = jnp.zeros((NPAD, 16), f32).at[:NNODES, :3].set(x)
    zeros16 = jnp.zeros((NPAD, 16), f32)
    ones_payload = jnp.ones((CHUNK_E, 16), f32)

    def pad16(w, b):
        wp = jnp.zeros((16, 16), f32).at[:w.shape[0], :w.shape[1]].set(w)
        bp = jnp.zeros((16,), f32).at[:b.shape[0]].set(b)
        return (jnp.kron(jnp.eye(8, dtype=f32), wp),
                jnp.tile(bp, 8).reshape(1, 128))

    wblk1, brep1 = pad16(W1, b1)
    wblkh0, breph0 = pad16(Wh0, bh0)
    wblkh1, breph1 = pad16(Wh1, bh1)
    w3p = jnp.zeros((16, 128), f32).at[:15, :120].set(W3)
    b3p = jnp.zeros((1, 128), f32).at[0, :120].set(b3)
    wcp = jnp.zeros((128, 8), f32).at[:120, :2].set(Wc)
    bcp = jnp.zeros((1, 8), f32).at[0, :2].set(bc)

    degp = _sc_deg(dst1d, ones_payload, zeros16)
    degp128 = degp.reshape(2 * NP8, 128)

    nblk = NP8 // _BR
    dinv128, g128 = pl.pallas_call(
        _tc_prep_body,
        grid=(nblk,),
        in_specs=[_row_spec(_BR, 128), _row_spec(_BR, 128, nblk),
                  _row_spec(_BR, 128)],
        out_specs=[_row_spec(_BR, 128), _row_spec(_BR, 128)],
        out_shape=[jax.ShapeDtypeStruct((NP8, 128), f32),
                   jax.ShapeDtypeStruct((NP8, 128), f32)],
    )(degp128, degp128, xpad.reshape(NP8, 128))

    for wblk, brep in ((wblk1, brep1), (wblkh0, breph0), (wblkh1, breph1)):
        y = _sc_agg(src1d, dst1d, g128.reshape(NPAD, 16))
        y128 = y.reshape(2 * NP8, 128)
        g128 = pl.pallas_call(
            _tc_layer_body,
            grid=(nblk,),
            in_specs=[_fixed_spec((128, 128)), _fixed_spec((1, 128)),
                      _row_spec(_BR, 128), _row_spec(_BR, 128, nblk),
                      _row_spec(_BR, 128), _row_spec(_BR, 128)],
            out_specs=_row_spec(_BR, 128),
            out_shape=jax.ShapeDtypeStruct((NP8, 128), f32),
        )(wblk, brep, y128, y128, g128, dinv128)

    y = _sc_agg(src1d, dst1d, g128.reshape(NPAD, 16))
    nblk2 = NPAD // _BR2
    hfull, ofull = pl.pallas_call(
        _tc_final_body,
        grid=(nblk2,),
        in_specs=[_fixed_spec((16, 128)), _fixed_spec((1, 128)),
                  _fixed_spec((128, 8)), _fixed_spec((1, 8)),
                  _row_spec(_BR2, 16), _row_spec(_BR2, 16, nblk2),
                  _row_spec(_BR2, 16), _row_spec(_BR2, 16)],
        out_specs=[_row_spec(_BR2, 128), _row_spec(_BR2, 8)],
        out_shape=[jax.ShapeDtypeStruct((NPAD, 128), f32),
                   jax.ShapeDtypeStruct((NPAD, 8), f32)],
    )(w3p, b3p, wcp, bcp, y, y, g128.reshape(NPAD, 16),
      dinv128.reshape(NPAD, 16))

    return (ofull[:NNODES, :2], hfull[:NNODES, :120])


# speedup vs baseline: 1.1142x; 1.0119x over previous
import functools

import jax
import jax.numpy as jnp
from jax import lax
from jax.experimental import pallas as pl
from jax.experimental.pallas import tpu as pltpu
from jax.experimental.pallas import tpu_sc as plsc

NNODES = 100000
NEDGES = 1600000
NPAD = 100352
NP8 = NPAD // 8
NTILES = 32
SLICE = NPAD // 16
CHUNK_E = 768
NOUT = 66
PAIRS = (NOUT - 2) // 2
EPAD = NTILES * CHUNK_E * NOUT
AGG_NOUT0 = 96
AGG_NOUT1 = 36

_MESH = plsc.VectorSubcoreMesh(
    core_axis_name="c", subcore_axis_name="s", num_cores=2, num_subcores=16)

_SC_PARAMS = pltpu.CompilerParams(use_tc_tiling_on_sc=False)


@functools.partial(
    pl.kernel,
    out_type=jax.ShapeDtypeStruct((2 * NPAD, 16), jnp.float32),
    mesh=_MESH,
    scratch_types=[
        pltpu.VMEM((CHUNK_E,), jnp.int32),
        pltpu.VMEM((CHUNK_E,), jnp.int32),
        pltpu.VMEM((CHUNK_E, 16), jnp.float32),
        pltpu.VMEM_SHARED((NPAD, 16), jnp.float32),
        pltpu.SemaphoreType.DMA,
        pltpu.SemaphoreType.DMA,
    ],
    compiler_params=_SC_PARAMS,
)
def _sc_deg(dst_hbm, ones_hbm, zeros_hbm, out_hbm, db0, db1, obuf, acc,
            ssem, isem):
    c = lax.axis_index("c")
    s = lax.axis_index("s")
    wid = s * 2 + c
    r0 = s * SLICE
    pltpu.sync_copy(zeros_hbm.at[pl.ds(r0, SLICE)], acc.at[pl.ds(r0, SLICE)])
    pltpu.sync_copy(ones_hbm, obuf)
    plsc.subcore_barrier()
    base = wid * (CHUNK_E * NOUT)

    def load_idx(i, db):
        pltpu.async_copy(dst_hbm.at[pl.ds(base + i * CHUNK_E, CHUNK_E)],
                         db, isem)

    def wait_idx():
        pltpu.make_async_copy(dst_hbm.at[pl.ds(0, CHUNK_E)], db0, isem).wait()

    def fire_scatters(db):
        pltpu.async_copy(obuf, acc.at[db], ssem, add=True)

    def wait_scatters():
        pltpu.make_async_copy(ones_hbm, obuf, ssem).wait()

    load_idx(0, db0)
    wait_idx()
    load_idx(1, db1)
    fire_scatters(db0)

    @pl.loop(0, PAIRS)
    def _pair(k):
        i1 = 2 * k + 1
        wait_idx()
        wait_scatters()
        load_idx(i1 + 1, db0)
        fire_scatters(db1)
        wait_idx()
        wait_scatters()
        load_idx(i1 + 2, db1)
        fire_scatters(db0)

    wait_idx()
    wait_scatters()
    fire_scatters(db1)
    wait_scatters()

    plsc.subcore_barrier()
    pltpu.sync_copy(acc.at[pl.ds(r0, SLICE)],
                    out_hbm.at[pl.ds(c * NPAD + r0, SLICE)])


@functools.partial(
    pl.kernel,
    out_type=jax.ShapeDtypeStruct((2 * NPAD, 16), jnp.float32),
    mesh=_MESH,
    scratch_types=[
        pltpu.VMEM((CHUNK_E,), jnp.int32),
        pltpu.VMEM((CHUNK_E,), jnp.int32),
        pltpu.VMEM((CHUNK_E,), jnp.int32),
        pltpu.VMEM((CHUNK_E,), jnp.int32),
        pltpu.VMEM((CHUNK_E, 16), jnp.float32),
        pltpu.VMEM((CHUNK_E, 16), jnp.float32),
        pltpu.VMEM_SHARED((NPAD, 16), jnp.float32),
        pltpu.SemaphoreType.DMA,
        pltpu.SemaphoreType.DMA,
        pltpu.SemaphoreType.DMA,
    ],
    compiler_params=_SC_PARAMS,
)
def _sc_agg(src_hbm, dst_hbm, g_hbm, out_hbm, sb0, db0, sb1, db1, rb0, rb1,
            acc, gsem, ssem, isem):
    c = lax.axis_index("c")
    s = lax.axis_index("s")
    wid = s * 2 + c
    r0 = s * SLICE
    pltpu.sync_copy(g_hbm.at[pl.ds(r0, SLICE)], acc.at[pl.ds(r0, SLICE)])
    plsc.subcore_barrier()
    base = jnp.where(c == 0, s * AGG_NOUT0,
                     16 * AGG_NOUT0 + s * AGG_NOUT1) * CHUNK_E
    pairs = jnp.where(c == 0, (AGG_NOUT0 - 2) // 2, (AGG_NOUT1 - 2) // 2)

    def load_idx(i, sb, db):
        pltpu.async_copy(src_hbm.at[pl.ds(base + i * CHUNK_E, CHUNK_E)],
                         sb, isem)
        pltpu.async_copy(dst_hbm.at[pl.ds(base + i * CHUNK_E, CHUNK_E)],
                         db, isem)

    def wait_idx():
        for _ in range(2):
            pltpu.make_async_copy(src_hbm.at[pl.ds(0, CHUNK_E)], sb0,
                                  isem).wait()

    def fire_gathers(sb, rb):
        pltpu.async_copy(g_hbm.at[sb], rb, gsem)

    def wait_gathers(rb):
        pltpu.make_async_copy(g_hbm.at[pl.ds(0, CHUNK_E)], rb, gsem).wait()

    def fire_scatters(db, rb):
        pltpu.async_copy(rb, acc.at[db], ssem, add=True)

    def wait_scatters(rb):
        pltpu.make_async_copy(g_hbm.at[pl.ds(0, CHUNK_E)], rb, ssem).wait()

    load_idx(0, sb0, db0)
    wait_idx()
    fire_gathers(sb0, rb0)
    load_idx(1, sb1, db1)
    wait_gathers(rb0)
    fire_scatters(db0, rb0)

    @pl.loop(0, pairs)
    def _pair(k):
        i1 = 2 * k + 1
        wait_idx()
        fire_gathers(sb1, rb1)
        wait_scatters(rb0)
        load_idx(i1 + 1, sb0, db0)
        wait_gathers(rb1)
        fire_scatters(db1, rb1)
        wait_idx()
        fire_gathers(sb0, rb0)
        wait_scatters(rb1)
        load_idx(i1 + 2, sb1, db1)
        wait_gathers(rb0)
        fire_scatters(db0, rb0)

    wait_idx()
    fire_gathers(sb1, rb1)
    wait_scatters(rb0)
    wait_gathers(rb1)
    fire_scatters(db1, rb1)
    wait_scatters(rb1)

    plsc.subcore_barrier()
    pltpu.sync_copy(acc.at[pl.ds(r0, SLICE)],
                    out_hbm.at[pl.ds(c * NPAD + r0, SLICE)])


_BR = 1568


def _tc_prep_body(d0, d1, xp, dinv_out, g1_out):
    dinv = 1.0 / jnp.sqrt(d0[...] + d1[...] + 1.0)
    dinv_out[...] = dinv
    g1_out[...] = dinv * xp[...]


def _tc_layer_body(wblk, brep, y0, y1, g, dinv, gnext_out):
    z = dinv[...] * (y0[...] + y1[...] - g[...])
    h = jnp.tanh(
        jnp.dot(z, wblk[...], preferred_element_type=jnp.float32,
                precision=lax.Precision.HIGHEST) + brep[...])
    gnext_out[...] = dinv[...] * h


_BR2 = 3584


def _tc_final_body(w3, b3, wc, bc2, y0, y1, g, dinv, h_out, o_out):
    z = dinv[...] * (y0[...] + y1[...] - g[...])
    h = jnp.tanh(
        jnp.dot(z, w3[...], preferred_element_type=jnp.float32,
                precision=lax.Precision.HIGHEST) + b3[...])
    h_out[...] = h
    o_out[...] = (
        jnp.dot(h, wc[...], preferred_element_type=jnp.float32,
                precision=lax.Precision.HIGHEST) + bc2[...])


def _row_spec(br, ncols, offset_blocks=0):
    return pl.BlockSpec((br, ncols), lambda i, o=offset_blocks: (i + o, 0))


def _fixed_spec(shape):
    return pl.BlockSpec(shape, lambda i: (0, 0))


def kernel(x, edge_index, W1, b1, Wh0, bh0, Wh1, bh1, W3, b3, Wc, bc):
    f32 = jnp.float32
    src = edge_index[0]
    dst = edge_index[1]
    pad = EPAD - NEDGES
    padv = jnp.full((pad,), NNODES, jnp.int32)
    src1d = jnp.concatenate([src, padv])
    dst1d = jnp.concatenate([dst, padv])
    xpad = jnp.zeros((NPAD, 16), f32).at[:NNODES, :3].set(x)
    zeros16 = jnp.zeros((NPAD, 16), f32)
    ones_payload = jnp.ones((CHUNK_E, 16), f32)

    def pad16(w, b):
        wp = jnp.zeros((16, 16), f32).at[:w.shape[0], :w.shape[1]].set(w)
        bp = jnp.zeros((16,), f32).at[:b.shape[0]].set(b)
        return (jnp.kron(jnp.eye(8, dtype=f32), wp),
                jnp.tile(bp, 8).reshape(1, 128))

    wblk1, brep1 = pad16(W1, b1)
    wblkh0, breph0 = pad16(Wh0, bh0)
    wblkh1, breph1 = pad16(Wh1, bh1)
    w3p = jnp.zeros((16, 128), f32).at[:15, :120].set(W3)
    b3p = jnp.zeros((1, 128), f32).at[0, :120].set(b3)
    wcp = jnp.zeros((128, 8), f32).at[:120, :2].set(Wc)
    bcp = jnp.zeros((1, 8), f32).at[0, :2].set(bc)

    degp = _sc_deg(dst1d, ones_payload, zeros16)
    degp128 = degp.reshape(2 * NP8, 128)

    nblk = NP8 // _BR
    dinv128, g128 = pl.pallas_call(
        _tc_prep_body,
        grid=(nblk,),
        in_specs=[_row_spec(_BR, 128), _row_spec(_BR, 128, nblk),
                  _row_spec(_BR, 128)],
        out_specs=[_row_spec(_BR, 128), _row_spec(_BR, 128)],
        out_shape=[jax.ShapeDtypeStruct((NP8, 128), f32),
                   jax.ShapeDtypeStruct((NP8, 128), f32)],
    )(degp128, degp128, xpad.reshape(NP8, 128))

    for wblk, brep in ((wblk1, brep1), (wblkh0, breph0), (wblkh1, breph1)):
        y = _sc_agg(src1d, dst1d, g128.reshape(NPAD, 16))
        y128 = y.reshape(2 * NP8, 128)
        g128 = pl.pallas_call(
            _tc_layer_body,
            grid=(nblk,),
            in_specs=[_fixed_spec((128, 128)), _fixed_spec((1, 128)),
                      _row_spec(_BR, 128), _row_spec(_BR, 128, nblk),
                      _row_spec(_BR, 128), _row_spec(_BR, 128)],
            out_specs=_row_spec(_BR, 128),
            out_shape=jax.ShapeDtypeStruct((NP8, 128), f32),
        )(wblk, brep, y128, y128, g128, dinv128)

    y = _sc_agg(src1d, dst1d, g128.reshape(NPAD, 16))
    nblk2 = NPAD // _BR2
    hfull, ofull = pl.pallas_call(
        _tc_final_body,
        grid=(nblk2,),
        in_specs=[_fixed_spec((16, 128)), _fixed_spec((1, 128)),
                  _fixed_spec((128, 8)), _fixed_spec((1, 8)),
                  _row_spec(_BR2, 16), _row_spec(_BR2, 16, nblk2),
                  _row_spec(_BR2, 16), _row_spec(_BR2, 16)],
        out_specs=[_row_spec(_BR2, 128), _row_spec(_BR2, 8)],
        out_shape=[jax.ShapeDtypeStruct((NPAD, 128), f32),
                   jax.ShapeDtypeStruct((NPAD, 8), f32)],
    )(w3p, b3p, wcp, bcp, y, y, g128.reshape(NPAD, 16),
      dinv128.reshape(NPAD, 16))

    return (ofull[:NNODES, :2], hfull[:NNODES, :120])


# speedup vs baseline: 1.1716x; 1.0515x over previous
import functools

import jax
import jax.numpy as jnp
from jax import lax
from jax.experimental import pallas as pl
from jax.experimental.pallas import tpu as pltpu
from jax.experimental.pallas import tpu_sc as plsc

NNODES = 100000
NEDGES = 1600000
NPAD = 100352
NP8 = NPAD // 8
NTILES = 32
SLICE = NPAD // 16
CHUNK_E = 768
NOUT = 66
PAIRS = (NOUT - 2) // 2
EPAD = NTILES * CHUNK_E * NOUT
AGG_NOUT0 = 96
AGG_NOUT1 = 36

_MESH = plsc.VectorSubcoreMesh(
    core_axis_name="c", subcore_axis_name="s", num_cores=2, num_subcores=16)

_SC_PARAMS = pltpu.CompilerParams(use_tc_tiling_on_sc=False)


@functools.partial(
    pl.kernel,
    out_type=jax.ShapeDtypeStruct((2 * NPAD, 16), jnp.float32),
    mesh=_MESH,
    scratch_types=[
        pltpu.VMEM((CHUNK_E,), jnp.int32),
        pltpu.VMEM((CHUNK_E,), jnp.int32),
        pltpu.VMEM((CHUNK_E, 16), jnp.float32),
        pltpu.VMEM_SHARED((NPAD, 16), jnp.float32),
        pltpu.SemaphoreType.DMA,
        pltpu.SemaphoreType.DMA,
    ],
    compiler_params=_SC_PARAMS,
)
def _sc_deg(dst_hbm, ones_hbm, zeros_hbm, out_hbm, db0, db1, obuf, acc,
            ssem, isem):
    c = lax.axis_index("c")
    s = lax.axis_index("s")
    wid = s * 2 + c
    r0 = s * SLICE
    pltpu.sync_copy(zeros_hbm.at[pl.ds(r0, SLICE)], acc.at[pl.ds(r0, SLICE)])
    pltpu.sync_copy(ones_hbm, obuf)
    plsc.subcore_barrier()
    base = wid * (CHUNK_E * NOUT)

    def load_idx(i, db):
        pltpu.async_copy(dst_hbm.at[pl.ds(base + i * CHUNK_E, CHUNK_E)],
                         db, isem)

    def wait_idx():
        pltpu.make_async_copy(dst_hbm.at[pl.ds(0, CHUNK_E)], db0, isem).wait()

    def fire_scatters(db):
        pltpu.async_copy(obuf, acc.at[db], ssem, add=True)

    def wait_scatters():
        pltpu.make_async_copy(ones_hbm, obuf, ssem).wait()

    load_idx(0, db0)
    wait_idx()
    load_idx(1, db1)
    fire_scatters(db0)

    @pl.loop(0, PAIRS)
    def _pair(k):
        i1 = 2 * k + 1
        wait_idx()
        wait_scatters()
        load_idx(i1 + 1, db0)
        fire_scatters(db1)
        wait_idx()
        wait_scatters()
        load_idx(i1 + 2, db1)
        fire_scatters(db0)

    wait_idx()
    wait_scatters()
    fire_scatters(db1)
    wait_scatters()

    plsc.subcore_barrier()
    pltpu.sync_copy(acc.at[pl.ds(r0, SLICE)],
                    out_hbm.at[pl.ds(c * NPAD + r0, SLICE)])


@functools.partial(
    pl.kernel,
    out_type=jax.ShapeDtypeStruct((2 * NPAD, 16), jnp.float32),
    mesh=_MESH,
    scratch_types=[
        pltpu.VMEM((CHUNK_E,), jnp.int32),
        pltpu.VMEM((CHUNK_E,), jnp.int32),
        pltpu.VMEM((CHUNK_E,), jnp.int32),
        pltpu.VMEM((CHUNK_E,), jnp.int32),
        pltpu.VMEM((CHUNK_E, 16), jnp.float32),
        pltpu.VMEM((CHUNK_E, 16), jnp.float32),
        pltpu.VMEM_SHARED((NPAD, 16), jnp.float32),
        pltpu.SemaphoreType.DMA,
        pltpu.SemaphoreType.DMA,
        pltpu.SemaphoreType.DMA,
    ],
    compiler_params=_SC_PARAMS,
)
def _sc_agg(src_hbm, dst_hbm, g_hbm, out_hbm, sb0, db0, sb1, db1, rb0, rb1,
            acc, gsem, ssem, isem):
    c = lax.axis_index("c")
    s = lax.axis_index("s")
    wid = s * 2 + c
    r0 = s * SLICE
    pltpu.sync_copy(g_hbm.at[pl.ds(r0, SLICE)], acc.at[pl.ds(r0, SLICE)])
    plsc.subcore_barrier()
    base = jnp.where(c == 0, s * AGG_NOUT0,
                     16 * AGG_NOUT0 + s * AGG_NOUT1) * CHUNK_E
    pairs = jnp.where(c == 0, (AGG_NOUT0 - 2) // 2, (AGG_NOUT1 - 2) // 2)

    def load_idx(i, sb, db):
        pltpu.async_copy(src_hbm.at[pl.ds(base + i * CHUNK_E, CHUNK_E)],
                         sb, isem)
        pltpu.async_copy(dst_hbm.at[pl.ds(base + i * CHUNK_E, CHUNK_E)],
                         db, isem)

    def wait_idx():
        for _ in range(2):
            pltpu.make_async_copy(src_hbm.at[pl.ds(0, CHUNK_E)], sb0,
                                  isem).wait()

    def fire_gathers(sb, rb):
        pltpu.async_copy(g_hbm.at[sb], rb, gsem)

    def wait_gathers(rb):
        pltpu.make_async_copy(g_hbm.at[pl.ds(0, CHUNK_E)], rb, gsem).wait()

    def fire_scatters(db, rb):
        pltpu.async_copy(rb, acc.at[db], ssem, add=True)

    def wait_scatters(rb):
        pltpu.make_async_copy(g_hbm.at[pl.ds(0, CHUNK_E)], rb, ssem).wait()

    load_idx(0, sb0, db0)
    wait_idx()
    fire_gathers(sb0, rb0)
    load_idx(1, sb1, db1)
    wait_gathers(rb0)
    fire_scatters(db0, rb0)

    @pl.loop(0, pairs)
    def _pair(k):
        i1 = 2 * k + 1
        wait_idx()
        fire_gathers(sb1, rb1)
        wait_scatters(rb0)
        load_idx(i1 + 1, sb0, db0)
        wait_gathers(rb1)
        fire_scatters(db1, rb1)
        wait_idx()
        fire_gathers(sb0, rb0)
        wait_scatters(rb1)
        load_idx(i1 + 2, sb1, db1)
        wait_gathers(rb0)
        fire_scatters(db0, rb0)

    wait_idx()
    fire_gathers(sb1, rb1)
    wait_scatters(rb0)
    wait_gathers(rb1)
    fire_scatters(db1, rb1)
    wait_scatters(rb1)

    plsc.subcore_barrier()
    pltpu.sync_copy(acc.at[pl.ds(r0, SLICE)],
                    out_hbm.at[pl.ds(c * NPAD + r0, SLICE)])


_BR = 1568


def _tc_prep_body(d0, d1, xp, dinv_out, g1_out):
    dinv = 1.0 / jnp.sqrt(d0[...] + d1[...] + 1.0)
    dinv_out[...] = dinv
    g1_out[...] = dinv * xp[...]


def _tc_layer_body(wblk, brep, y0, y1, g, dinv, gnext_out):
    z = dinv[...] * (y0[...] + y1[...] - g[...])
    h = jnp.tanh(
        jnp.dot(z, wblk[...], preferred_element_type=jnp.float32) + brep[...])
    gnext_out[...] = dinv[...] * h


_BR2 = 3584


def _tc_final_body(w3, b3, wc, bc2, y0, y1, g, dinv, h_out, o_out):
    z = dinv[...] * (y0[...] + y1[...] - g[...])
    h = jnp.tanh(
        jnp.dot(z, w3[...], preferred_element_type=jnp.float32) + b3[...])
    h_out[...] = h
    o_out[...] = (
        jnp.dot(h, wc[...], preferred_element_type=jnp.float32) + bc2[...])


def _row_spec(br, ncols, offset_blocks=0):
    return pl.BlockSpec((br, ncols), lambda i, o=offset_blocks: (i + o, 0))


def _fixed_spec(shape):
    return pl.BlockSpec(shape, lambda i: (0, 0))


def kernel(x, edge_index, W1, b1, Wh0, bh0, Wh1, bh1, W3, b3, Wc, bc):
    f32 = jnp.float32
    src = edge_index[0]
    dst = edge_index[1]
    pad = EPAD - NEDGES
    padv = jnp.full((pad,), NNODES, jnp.int32)
    src1d = jnp.concatenate([src, padv])
    dst1d = jnp.concatenate([dst, padv])
    xpad = jnp.zeros((NPAD, 16), f32).at[:NNODES, :3].set(x)
    zeros16 = jnp.zeros((NPAD, 16), f32)
    ones_payload = jnp.ones((CHUNK_E, 16), f32)

    def pad16(w, b):
        wp = jnp.zeros((16, 16), f32).at[:w.shape[0], :w.shape[1]].set(w)
        bp = jnp.zeros((16,), f32).at[:b.shape[0]].set(b)
        return (jnp.kron(jnp.eye(8, dtype=f32), wp),
                jnp.tile(bp, 8).reshape(1, 128))

    wblk1, brep1 = pad16(W1, b1)
    wblkh0, breph0 = pad16(Wh0, bh0)
    wblkh1, breph1 = pad16(Wh1, bh1)
    w3p = jnp.zeros((16, 128), f32).at[:15, :120].set(W3)
    b3p = jnp.zeros((1, 128), f32).at[0, :120].set(b3)
    wcp = jnp.zeros((128, 8), f32).at[:120, :2].set(Wc)
    bcp = jnp.zeros((1, 8), f32).at[0, :2].set(bc)

    degp = _sc_deg(dst1d, ones_payload, zeros16)
    degp128 = degp.reshape(2 * NP8, 128)

    nblk = NP8 // _BR
    dinv128, g128 = pl.pallas_call(
        _tc_prep_body,
        grid=(nblk,),
        in_specs=[_row_spec(_BR, 128), _row_spec(_BR, 128, nblk),
                  _row_spec(_BR, 128)],
        out_specs=[_row_spec(_BR, 128), _row_spec(_BR, 128)],
        out_shape=[jax.ShapeDtypeStruct((NP8, 128), f32),
                   jax.ShapeDtypeStruct((NP8, 128), f32)],
    )(degp128, degp128, xpad.reshape(NP8, 128))

    for wblk, brep in ((wblk1, brep1), (wblkh0, breph0), (wblkh1, breph1)):
        y = _sc_agg(src1d, dst1d, g128.reshape(NPAD, 16))
        y128 = y.reshape(2 * NP8, 128)
        g128 = pl.pallas_call(
            _tc_layer_body,
            grid=(nblk,),
            in_specs=[_fixed_spec((128, 128)), _fixed_spec((1, 128)),
                      _row_spec(_BR, 128), _row_spec(_BR, 128, nblk),
                      _row_spec(_BR, 128), _row_spec(_BR, 128)],
            out_specs=_row_spec(_BR, 128),
            out_shape=jax.ShapeDtypeStruct((NP8, 128), f32),
        )(wblk, brep, y128, y128, g128, dinv128)

    y = _sc_agg(src1d, dst1d, g128.reshape(NPAD, 16))
    nblk2 = NPAD // _BR2
    hfull, ofull = pl.pallas_call(
        _tc_final_body,
        grid=(nblk2,),
        in_specs=[_fixed_spec((16, 128)), _fixed_spec((1, 128)),
                  _fixed_spec((128, 8)), _fixed_spec((1, 8)),
                  _row_spec(_BR2, 16), _row_spec(_BR2, 16, nblk2),
                  _row_spec(_BR2, 16), _row_spec(_BR2, 16)],
        out_specs=[_row_spec(_BR2, 128), _row_spec(_BR2, 8)],
        out_shape=[jax.ShapeDtypeStruct((NPAD, 128), f32),
                   jax.ShapeDtypeStruct((NPAD, 8), f32)],
    )(w3p, b3p, wcp, bcp, y, y, g128.reshape(NPAD, 16),
      dinv128.reshape(NPAD, 16))

    return (ofull[:NNODES, :2], hfull[:NNODES, :120])
